# Initial kernel scaffold; baseline (speedup 1.0000x reference)
import jax
import jax.numpy as jnp
from jax.experimental import pallas as pl


def kernel(features, pos, rgb, target):
    raise NotImplementedError("write your pallas kernel here")


# speedup vs baseline: 10.8445x; 10.8445x over previous
import jax
import jax.numpy as jnp
from jax.experimental import pallas as pl

_N = 10000
_F = 64
_K = 16
_R = 400
_SIGMA_G = 0.1
_LAMBDA_TEX = 5.0


def _normalize_kernel(f_ref, c_ref):
    f = f_ref[...]
    nrm = jnp.sqrt(jnp.sum(f * f, axis=1, keepdims=True))
    z = f / jnp.maximum(nrm, 1e-12)
    c_ref[:, 0:_F] = z
    c_ref[:, _F:2 * _F] = z * z


def _loss_kernel(aux_row_ref, aux_col_ref, c_full_ref, c_row_ref, out_ref):
    i = pl.program_id(0)

    px = aux_row_ref[:, 0:1]
    py = aux_row_ref[:, 1:2]
    pz = aux_row_ref[:, 2:3]
    cx = aux_col_ref[0:1, :]
    cy = aux_col_ref[1:2, :]
    cz = aux_col_ref[2:3, :]

    dx = px - cx
    dy = py - cy
    dz = pz - cz
    d2 = dx * dx + dy * dy + dz * dz
    dist = jnp.sqrt(jnp.maximum(d2, 0.0))

    col = jax.lax.broadcasted_iota(jnp.int32, (1, _N), 1)
    rowg = i * _R + jax.lax.broadcasted_iota(jnp.int32, (_R, 1), 0)
    inf = jnp.float32(jnp.inf)
    dorig = jnp.where(col == rowg, inf, dist)

    bigi = jnp.int32(2 ** 30)

    def body(_, carry):
        dw, _, _ = carry
        m = jnp.min(dw, axis=1, keepdims=True)
        jj = jnp.min(jnp.where(dw == m, col, bigi), axis=1, keepdims=True)
        dw = jnp.where(col == jj, inf, dw)
        return dw, m, jj

    init = (dorig,
            jnp.zeros((_R, 1), jnp.float32),
            jnp.zeros((_R, 1), jnp.int32))
    _, thr, jstar = jax.lax.fori_loop(0, _K, body, init)

    sel = (dorig < thr) | ((dorig == thr) & (col <= jstar))

    same = aux_row_ref[:, 6:7] == aux_col_ref[6:7, :]
    rx = aux_row_ref[:, 3:4] - aux_col_ref[3:4, :]
    ry = aux_row_ref[:, 4:5] - aux_col_ref[4:5, :]
    rz = aux_row_ref[:, 5:6] - aux_col_ref[5:6, :]
    drgb = rx * rx + ry * ry + rz * rz
    w = jnp.exp(d2 * (-1.0 / (2.0 * _SIGMA_G ** 2))) * jnp.exp(drgb * (-_LAMBDA_TEX))
    wmat = jnp.where(sel & same, w, 0.0)

    roww = jnp.sum(wmat, axis=1, keepdims=True)
    a = jnp.dot(wmat, c_full_ref[...],
                preferred_element_type=jnp.float32)
    a1 = a[:, 0:_F]
    b = jnp.sum(a[:, _F:2 * _F], axis=1, keepdims=True)
    zr = c_row_ref[:, 0:_F]
    qr = jnp.sum(c_row_ref[:, _F:2 * _F], axis=1, keepdims=True)
    li = qr * roww + b - 2.0 * jnp.sum(zr * a1, axis=1, keepdims=True)
    part = jnp.sum(li, axis=0, keepdims=True)

    @pl.when(i == 0)
    def _():
        out_ref[...] = jnp.zeros((1, 1), jnp.float32)

    out_ref[...] += part


def kernel(features, pos, rgb, target):
    n = pos.shape[0]
    c = pl.pallas_call(
        _normalize_kernel,
        out_shape=jax.ShapeDtypeStruct((n, 2 * _F), jnp.float32),
    )(features)

    aux_row = jnp.concatenate(
        [pos, rgb, target.astype(jnp.float32)[:, None],
         jnp.zeros((n, 1), jnp.float32)], axis=1)
    aux_col = aux_row.T

    grid = n // _R
    total = pl.pallas_call(
        _loss_kernel,
        grid=(grid,),
        in_specs=[
            pl.BlockSpec((_R, 8), lambda i: (i, 0)),
            pl.BlockSpec((8, n), lambda i: (0, 0)),
            pl.BlockSpec((n, 2 * _F), lambda i: (0, 0)),
            pl.BlockSpec((_R, 2 * _F), lambda i: (i, 0)),
        ],
        out_specs=pl.BlockSpec((1, 1), lambda i: (0, 0)),
        out_shape=jax.ShapeDtypeStruct((1, 1), jnp.float32),
    )(aux_row, aux_col, c, c)

    return total[0, 0] / jnp.float32(n)
